# baseline (speedup 1.0000x reference)
import functools

import jax
import jax.numpy as jnp
from jax import lax
from jax.experimental import pallas as pl
from jax.experimental.pallas import tpu as pltpu
from jax.experimental.pallas import tpu_sc as plsc


def _sc_gather(users, movies, user_table, movie_table):
    B = users.shape[0]
    E = user_table.shape[1]
    info = plsc.get_sparse_core_info()
    NC, NS = info.num_cores, info.num_subcores
    NW = NC * NS
    BPW = B // NW
    G = BPW // 16

    mesh = plsc.VectorSubcoreMesh(core_axis_name="c", subcore_axis_name="s")

    @functools.partial(
        pl.kernel,
        mesh=mesh,
        out_type=[
            jax.ShapeDtypeStruct((B, E), jnp.float32),
            jax.ShapeDtypeStruct((B, E), jnp.float32),
        ],
        scratch_types=[
            pltpu.VMEM((BPW,), jnp.int32),
            pltpu.VMEM((BPW,), jnp.int32),
            pltpu.VMEM((BPW // 2, E), jnp.float32),
            pltpu.VMEM((BPW // 2, E), jnp.float32),
            pltpu.SemaphoreType.DMA,
        ],
    )
    def gather_kernel(u_idx_hbm, m_idx_hbm, utab_hbm, mtab_hbm,
                      u_out_hbm, m_out_hbm,
                      uidx_v, midx_v, urows_v, mrows_v, sem):
        wid = lax.axis_index("s") * NC + lax.axis_index("c")
        base = wid * BPW
        half = BPW // 2
        pltpu.sync_copy(u_idx_hbm.at[pl.ds(base, BPW)], uidx_v)
        pltpu.sync_copy(m_idx_hbm.at[pl.ds(base, BPW)], midx_v)

        def fire_group(h, g):
            uvec = uidx_v[pl.ds(h * half + g * 16, 16)]
            mvec = midx_v[pl.ds(h * half + g * 16, 16)]
            for j in range(16):
                pltpu.async_copy(utab_hbm.at[pl.ds(uvec[j], 1)],
                                 urows_v.at[pl.ds(g * 16 + j, 1)], sem)
                pltpu.async_copy(mtab_hbm.at[pl.ds(mvec[j], 1)],
                                 mrows_v.at[pl.ds(g * 16 + j, 1)], sem)

        def drain_group():
            for _ in range(32):
                pltpu.make_async_copy(utab_hbm.at[pl.ds(0, 1)],
                                      urows_v.at[pl.ds(0, 1)], sem).wait()

        for h in range(2):
            fire_group(h, 0)

            def body(g, _, h=h):
                fire_group(h, g)
                drain_group()
                return ()

            lax.fori_loop(1, G // 2, body, ())
            drain_group()

            pltpu.sync_copy(urows_v, u_out_hbm.at[pl.ds(base + h * half, half)])
            pltpu.sync_copy(mrows_v, m_out_hbm.at[pl.ds(base + h * half, half)])

    return gather_kernel(users, movies, user_table, movie_table)


def _mlp_body(u_ref, m_ref, w1u_ref, w1m_ref, b1_ref, w2_ref, b2_ref,
              w3_ref, b3_ref, out_ref):
    x = jnp.dot(u_ref[...], w1u_ref[...], preferred_element_type=jnp.float32)
    x = x + jnp.dot(m_ref[...], w1m_ref[...],
                    preferred_element_type=jnp.float32)
    h1 = jnp.maximum(x + b1_ref[...], 0.0)
    h2 = jnp.dot(h1, w2_ref[...], preferred_element_type=jnp.float32)
    h2 = jnp.maximum(h2 + b2_ref[...], 0.0)
    o = jnp.sum(h2 * w3_ref[...], axis=1) + b3_ref[0, 0]
    out_ref[...] = o


def _tc_mlp(u, m, W1, b1, W2, b2, W3, b3, blk=2048):
    B, E = u.shape
    H1 = W1.shape[0]
    H2 = W2.shape[0]
    w1u = W1[:, :E].T
    w1m = W1[:, E:].T
    w2t = W2.T
    b1r = b1.reshape(1, H1)
    b2r = b2.reshape(1, H2)
    w3r = W3.reshape(1, H2)
    b3r = b3.reshape(1, 1)

    grid = (B // blk,)
    full = lambda i: (0, 0)
    return pl.pallas_call(
        _mlp_body,
        grid=grid,
        in_specs=[
            pl.BlockSpec((blk, E), lambda i: (i, 0)),
            pl.BlockSpec((blk, E), lambda i: (i, 0)),
            pl.BlockSpec((E, H1), full),
            pl.BlockSpec((E, H1), full),
            pl.BlockSpec((1, H1), full),
            pl.BlockSpec((H1, H2), full),
            pl.BlockSpec((1, H2), full),
            pl.BlockSpec((1, H2), full),
            pl.BlockSpec((1, 1), full),
        ],
        out_specs=pl.BlockSpec((blk,), lambda i: (i,)),
        out_shape=jax.ShapeDtypeStruct((B,), jnp.float32),
    )(u, m, w1u, w1m, b1r, w2t, b2r, w3r, b3r)


def kernel(users, movies, user_table, movie_table, W1, b1, W2, b2, W3, b3):
    u, m = _sc_gather(users, movies, user_table, movie_table)
    return _tc_mlp(u, m, W1, b1, W2, b2, W3, b3)
